# baseline (speedup 1.0000x reference)
import functools

import jax
import jax.numpy as jnp
from jax import lax
from jax.experimental import pallas as pl
from jax.experimental.pallas import tpu as pltpu
from jax.experimental.pallas import tpu_sc as plsc

E = 320000
C = 128


CH = 2500
NBUF = 8


def _in_copy(hbm_ref, buf_ref, sem_ref, i, s, ch):
    return pltpu.make_async_copy(
        hbm_ref.at[pl.ds(i * ch, ch), :], buf_ref.at[s], sem_ref.at[s])


def _out_copy(out_hbm, outbuf, sem_ref, i, s):
    return pltpu.make_async_copy(
        outbuf.at[s], out_hbm.at[pl.ds(i, 1), :], sem_ref.at[s])


def _tc_body(nchunk, linv_hbm, adj_hbm, w_ref, b_ref, out_hbm,
             linv_buf, adj_buf, outbuf, sem_l, sem_a, sem_o):
    w = w_ref[...]
    bias = b_ref[0, 0]

    for s in range(NBUF):
        _in_copy(linv_hbm, linv_buf, sem_l, s, s, CH).start()
        _in_copy(adj_hbm, adj_buf, sem_a, s, s, CH).start()

    def outer(g, carry):
        for s in range(NBUF):
            i = g * NBUF + s
            _in_copy(linv_hbm, linv_buf, sem_l, i, s, CH).wait()
            _in_copy(adj_hbm, adj_buf, sem_a, i, s, CH).wait()

            h = jnp.maximum(adj_buf[s] * linv_buf[s], 0.0)
            res = jax.lax.dot_general(
                w, h, (((1,), (1,)), ((), ())),
                preferred_element_type=jnp.float32,
            ) + bias

            @pl.when(g > 0)
            def _wait_out():
                _out_copy(out_hbm, outbuf, sem_o, i - NBUF, s).wait()

            outbuf[s] = res

            @pl.when(i + NBUF < nchunk)
            def _next_in():
                _in_copy(linv_hbm, linv_buf, sem_l, i + NBUF, s, CH).start()
                _in_copy(adj_hbm, adj_buf, sem_a, i + NBUF, s, CH).start()

            _out_copy(out_hbm, outbuf, sem_o, i, s).start()
        return carry

    jax.lax.fori_loop(0, nchunk // NBUF, outer, 0)

    for s in range(NBUF):
        _out_copy(out_hbm, outbuf, sem_o, nchunk - NBUF + s, s).wait()


def _tc_part(Linv, adjacency, W, b2d, n_rows):
    nchunk = n_rows // CH
    out = pl.pallas_call(
        functools.partial(_tc_body, nchunk),
        in_specs=[
            pl.BlockSpec(memory_space=pltpu.MemorySpace.HBM),
            pl.BlockSpec(memory_space=pltpu.MemorySpace.HBM),
            pl.BlockSpec(memory_space=pltpu.MemorySpace.VMEM),
            pl.BlockSpec(memory_space=pltpu.MemorySpace.VMEM),
        ],
        out_specs=pl.BlockSpec(memory_space=pltpu.MemorySpace.HBM),
        out_shape=jax.ShapeDtypeStruct((nchunk, CH), jnp.float32),
        scratch_shapes=[
            pltpu.VMEM((NBUF, CH, C), jnp.float32),
            pltpu.VMEM((NBUF, CH, C), jnp.float32),
            pltpu.VMEM((NBUF, 1, CH), jnp.float32),
            pltpu.SemaphoreType.DMA((NBUF,)),
            pltpu.SemaphoreType.DMA((NBUF,)),
            pltpu.SemaphoreType.DMA((NBUF,)),
        ],
    )(Linv, adjacency, W, b2d)
    return out.reshape(n_rows)


NC = 2
NS = 16
NW = NC * NS
CHS = 80


def _sc_in_copy(hbm_ref, buf_ref, sem_ref, base, ch, chs):
    return pltpu.make_async_copy(
        hbm_ref.at[pl.ds(base + ch * chs, chs), :], buf_ref, sem_ref)


def _sc_compute_chunk(lbuf, abuf, w_v, b_v, out_v, out_base):
    bias = b_v[...]
    wsl = [w_v[pl.ds(16 * j, 16)] for j in range(C // 16)]

    def group(g, carry):
        row0 = g * 16
        idx_row = row0 + lax.iota(jnp.int32, 16)
        acc = jnp.zeros((16,), jnp.float32)
        for c in range(C):
            idx_col = jnp.full((16,), c, jnp.int32)
            va = plsc.load_gather(abuf, [idx_row, idx_col])
            vl = plsc.load_gather(lbuf, [idx_row, idx_col])
            acc = acc + jnp.maximum(va * vl, 0.0) * wsl[c // 16][c % 16]
        out_v[pl.ds(out_base + row0, 16)] = acc + bias
        return carry

    lax.fori_loop(0, CHS // 16, group, 0)


def _sc_part(Linv_sc, adjacency, W, b16, n_rows, row_off):
    rw = n_rows // NW
    nch = rw // CHS
    mesh = plsc.VectorSubcoreMesh(core_axis_name="c", subcore_axis_name="s")

    @functools.partial(
        pl.kernel, mesh=mesh,
        compiler_params=pltpu.CompilerParams(needs_layout_passes=False),
        out_type=jax.ShapeDtypeStruct((n_rows,), jnp.float32),
        scratch_types=[
            pltpu.VMEM((CHS, C), jnp.float32),
            pltpu.VMEM((CHS, C), jnp.float32),
            pltpu.VMEM((CHS, C), jnp.float32),
            pltpu.VMEM((CHS, C), jnp.float32),
            pltpu.VMEM((rw,), jnp.float32),
            pltpu.VMEM((C,), jnp.float32),
            pltpu.VMEM((16,), jnp.float32),
            pltpu.SemaphoreType.DMA,
            pltpu.SemaphoreType.DMA,
            pltpu.SemaphoreType.DMA,
            pltpu.SemaphoreType.DMA,
        ],
    )
    def sck(linv_hbm, adj_hbm, w_hbm, b_hbm, out_hbm,
            lbuf0, lbuf1, abuf0, abuf1, out_v, w_v, b_v,
            sl0, sl1, sa0, sa1):
        wid = lax.axis_index("s") * NC + lax.axis_index("c")
        base = row_off + wid * rw

        pltpu.sync_copy(w_hbm, w_v)
        pltpu.sync_copy(b_hbm, b_v)

        _sc_in_copy(linv_hbm, lbuf0, sl0, base, 0, CHS).start()
        _sc_in_copy(adj_hbm, abuf0, sa0, base, 0, CHS).start()
        _sc_in_copy(linv_hbm, lbuf1, sl1, base, 1, CHS).start()
        _sc_in_copy(adj_hbm, abuf1, sa1, base, 1, CHS).start()

        def pair(p, carry):
            ch0 = p * 2
            _sc_in_copy(linv_hbm, lbuf0, sl0, base, ch0, CHS).wait()
            _sc_in_copy(adj_hbm, abuf0, sa0, base, ch0, CHS).wait()
            _sc_compute_chunk(lbuf0, abuf0, w_v, b_v, out_v, ch0 * CHS)

            @pl.when(ch0 + 2 < nch)
            def _n0():
                _sc_in_copy(linv_hbm, lbuf0, sl0, base, ch0 + 2, CHS).start()
                _sc_in_copy(adj_hbm, abuf0, sa0, base, ch0 + 2, CHS).start()

            ch1 = ch0 + 1
            _sc_in_copy(linv_hbm, lbuf1, sl1, base, ch1, CHS).wait()
            _sc_in_copy(adj_hbm, abuf1, sa1, base, ch1, CHS).wait()
            _sc_compute_chunk(lbuf1, abuf1, w_v, b_v, out_v, ch1 * CHS)

            @pl.when(ch1 + 2 < nch)
            def _n1():
                _sc_in_copy(linv_hbm, lbuf1, sl1, base, ch1 + 2, CHS).start()
                _sc_in_copy(adj_hbm, abuf1, sa1, base, ch1 + 2, CHS).start()

            return carry

        lax.fori_loop(0, nch // 2, pair, 0)

        if nch % 2:
            ch = nch - 1
            _sc_in_copy(linv_hbm, lbuf0, sl0, base, ch, CHS).wait()
            _sc_in_copy(adj_hbm, abuf0, sa0, base, ch, CHS).wait()
            _sc_compute_chunk(lbuf0, abuf0, w_v, b_v, out_v, ch * CHS)

        pltpu.sync_copy(out_v, out_hbm.at[pl.ds(wid * rw, rw)])

    return sck(Linv_sc, adjacency, W.reshape(C), b16)


def kernel(x_e, Linv, adjacency, W, b):
    del x_e
    b16 = jnp.broadcast_to(b, (16,))
    out_sc = _sc_part(Linv, adjacency, W, b16, E, 0)
    return out_sc.reshape(E, 1)


# speedup vs baseline: 1.2659x; 1.2659x over previous
import functools

import jax
import jax.numpy as jnp
from jax import lax
from jax.experimental import pallas as pl
from jax.experimental.pallas import tpu as pltpu
from jax.experimental.pallas import tpu_sc as plsc

E = 320000
C = 128


CH = 2500
NBUF = 8


def _in_copy(hbm_ref, buf_ref, sem_ref, i, s, ch):
    return pltpu.make_async_copy(
        hbm_ref.at[pl.ds(i * ch, ch), :], buf_ref.at[s], sem_ref.at[s])


def _out_copy(out_hbm, outbuf, sem_ref, i, s):
    return pltpu.make_async_copy(
        outbuf.at[s], out_hbm.at[pl.ds(i, 1), :], sem_ref.at[s])


def _tc_body(nchunk, linv_hbm, adj_hbm, w_ref, b_ref, out_hbm,
             linv_buf, adj_buf, outbuf, sem_l, sem_a, sem_o):
    w = w_ref[...]
    bias = b_ref[0, 0]

    for s in range(NBUF):
        _in_copy(linv_hbm, linv_buf, sem_l, s, s, CH).start()
        _in_copy(adj_hbm, adj_buf, sem_a, s, s, CH).start()

    def outer(g, carry):
        for s in range(NBUF):
            i = g * NBUF + s
            _in_copy(linv_hbm, linv_buf, sem_l, i, s, CH).wait()
            _in_copy(adj_hbm, adj_buf, sem_a, i, s, CH).wait()

            h = jnp.maximum(adj_buf[s] * linv_buf[s], 0.0)
            res = jax.lax.dot_general(
                w, h, (((1,), (1,)), ((), ())),
                preferred_element_type=jnp.float32,
            ) + bias

            @pl.when(g > 0)
            def _wait_out():
                _out_copy(out_hbm, outbuf, sem_o, i - NBUF, s).wait()

            outbuf[s] = res

            @pl.when(i + NBUF < nchunk)
            def _next_in():
                _in_copy(linv_hbm, linv_buf, sem_l, i + NBUF, s, CH).start()
                _in_copy(adj_hbm, adj_buf, sem_a, i + NBUF, s, CH).start()

            _out_copy(out_hbm, outbuf, sem_o, i, s).start()
        return carry

    jax.lax.fori_loop(0, nchunk // NBUF, outer, 0)

    for s in range(NBUF):
        _out_copy(out_hbm, outbuf, sem_o, nchunk - NBUF + s, s).wait()


def _tc_part(Linv, adjacency, W, b2d, n_rows):
    nchunk = n_rows // CH
    out = pl.pallas_call(
        functools.partial(_tc_body, nchunk),
        in_specs=[
            pl.BlockSpec(memory_space=pltpu.MemorySpace.HBM),
            pl.BlockSpec(memory_space=pltpu.MemorySpace.HBM),
            pl.BlockSpec(memory_space=pltpu.MemorySpace.VMEM),
            pl.BlockSpec(memory_space=pltpu.MemorySpace.VMEM),
        ],
        out_specs=pl.BlockSpec(memory_space=pltpu.MemorySpace.HBM),
        out_shape=jax.ShapeDtypeStruct((nchunk, CH), jnp.float32),
        scratch_shapes=[
            pltpu.VMEM((NBUF, CH, C), jnp.float32),
            pltpu.VMEM((NBUF, CH, C), jnp.float32),
            pltpu.VMEM((NBUF, 1, CH), jnp.float32),
            pltpu.SemaphoreType.DMA((NBUF,)),
            pltpu.SemaphoreType.DMA((NBUF,)),
            pltpu.SemaphoreType.DMA((NBUF,)),
        ],
    )(Linv, adjacency, W, b2d)
    return out.reshape(n_rows)


NC = 2
NS = 16
NW = NC * NS
CHS = 80


def _sc_in_copy(hbm_ref, buf_ref, sem_ref, base, ch, chs):
    return pltpu.make_async_copy(
        hbm_ref.at[pl.ds(base + ch * chs, chs), :], buf_ref, sem_ref)


def _sc_compute_chunk(lbuf, abuf, w_v, b_v, out_v, out_base):
    bias = b_v[...]
    wsl = [w_v[pl.ds(16 * j, 16)] for j in range(C // 16)]
    NACC = 8

    @plsc.parallel_loop(0, CHS // 16)
    def group(g):
        row0 = g * 16
        idx_row = row0 + lax.iota(jnp.int32, 16)
        accs = [jnp.zeros((16,), jnp.float32) for _ in range(NACC)]
        for c in range(C):
            idx_col = jnp.full((16,), c, jnp.int32)
            va = plsc.load_gather(abuf, [idx_row, idx_col])
            vl = plsc.load_gather(lbuf, [idx_row, idx_col])
            accs[c % NACC] = accs[c % NACC] + (
                jnp.maximum(va * vl, 0.0) * wsl[c // 16][c % 16])
        acc = accs[0]
        for a in accs[1:]:
            acc = acc + a
        out_v[pl.ds(out_base + row0, 16)] = acc + bias


def _sc_part(Linv_sc, adjacency, W, b16, n_rows, row_off):
    rw = n_rows // NW
    nch = rw // CHS
    mesh = plsc.VectorSubcoreMesh(core_axis_name="c", subcore_axis_name="s")

    @functools.partial(
        pl.kernel, mesh=mesh,
        compiler_params=pltpu.CompilerParams(needs_layout_passes=False),
        out_type=jax.ShapeDtypeStruct((n_rows,), jnp.float32),
        scratch_types=[
            pltpu.VMEM((CHS, C), jnp.float32),
            pltpu.VMEM((CHS, C), jnp.float32),
            pltpu.VMEM((CHS, C), jnp.float32),
            pltpu.VMEM((CHS, C), jnp.float32),
            pltpu.VMEM((rw,), jnp.float32),
            pltpu.VMEM((C,), jnp.float32),
            pltpu.VMEM((16,), jnp.float32),
            pltpu.SemaphoreType.DMA,
            pltpu.SemaphoreType.DMA,
            pltpu.SemaphoreType.DMA,
            pltpu.SemaphoreType.DMA,
        ],
    )
    def sck(linv_hbm, adj_hbm, w_hbm, b_hbm, out_hbm,
            lbuf0, lbuf1, abuf0, abuf1, out_v, w_v, b_v,
            sl0, sl1, sa0, sa1):
        wid = lax.axis_index("s") * NC + lax.axis_index("c")
        base = row_off + wid * rw

        pltpu.sync_copy(w_hbm, w_v)
        pltpu.sync_copy(b_hbm, b_v)

        _sc_in_copy(linv_hbm, lbuf0, sl0, base, 0, CHS).start()
        _sc_in_copy(adj_hbm, abuf0, sa0, base, 0, CHS).start()
        _sc_in_copy(linv_hbm, lbuf1, sl1, base, 1, CHS).start()
        _sc_in_copy(adj_hbm, abuf1, sa1, base, 1, CHS).start()

        def pair(p, carry):
            ch0 = p * 2
            _sc_in_copy(linv_hbm, lbuf0, sl0, base, ch0, CHS).wait()
            _sc_in_copy(adj_hbm, abuf0, sa0, base, ch0, CHS).wait()
            _sc_compute_chunk(lbuf0, abuf0, w_v, b_v, out_v, ch0 * CHS)

            @pl.when(ch0 + 2 < nch)
            def _n0():
                _sc_in_copy(linv_hbm, lbuf0, sl0, base, ch0 + 2, CHS).start()
                _sc_in_copy(adj_hbm, abuf0, sa0, base, ch0 + 2, CHS).start()

            ch1 = ch0 + 1
            _sc_in_copy(linv_hbm, lbuf1, sl1, base, ch1, CHS).wait()
            _sc_in_copy(adj_hbm, abuf1, sa1, base, ch1, CHS).wait()
            _sc_compute_chunk(lbuf1, abuf1, w_v, b_v, out_v, ch1 * CHS)

            @pl.when(ch1 + 2 < nch)
            def _n1():
                _sc_in_copy(linv_hbm, lbuf1, sl1, base, ch1 + 2, CHS).start()
                _sc_in_copy(adj_hbm, abuf1, sa1, base, ch1 + 2, CHS).start()

            return carry

        lax.fori_loop(0, nch // 2, pair, 0)

        if nch % 2:
            ch = nch - 1
            _sc_in_copy(linv_hbm, lbuf0, sl0, base, ch, CHS).wait()
            _sc_in_copy(adj_hbm, abuf0, sa0, base, ch, CHS).wait()
            _sc_compute_chunk(lbuf0, abuf0, w_v, b_v, out_v, ch * CHS)

        pltpu.sync_copy(out_v, out_hbm.at[pl.ds(wid * rw, rw)])

    return sck(Linv_sc, adjacency, W.reshape(C), b16)


def kernel(x_e, Linv, adjacency, W, b):
    del x_e
    b16 = jnp.broadcast_to(b, (16,))
    out_sc = _sc_part(Linv, adjacency, W, b16, E, 0)
    return out_sc.reshape(E, 1)


# speedup vs baseline: 9.2193x; 7.2825x over previous
import functools

import jax
import jax.numpy as jnp
from jax import lax
from jax.experimental import pallas as pl
from jax.experimental.pallas import tpu as pltpu
from jax.experimental.pallas import tpu_sc as plsc

E = 320000
C = 128


CH = 2500
NBUF = 8


def _in_copy(hbm_ref, buf_ref, sem_ref, i, s, ch):
    return pltpu.make_async_copy(
        hbm_ref.at[pl.ds(i * ch, ch), :], buf_ref.at[s], sem_ref.at[s])


def _out_copy(out_hbm, outbuf, sem_ref, i, s):
    return pltpu.make_async_copy(
        outbuf.at[s], out_hbm.at[pl.ds(i, 1), :], sem_ref.at[s])


def _tc_body(nchunk, linv_hbm, adj_hbm, w_ref, b_ref, out_hbm,
             linv_buf, adj_buf, outbuf, sem_l, sem_a, sem_o):
    w = w_ref[...]
    bias = b_ref[0, 0]

    for s in range(NBUF):
        _in_copy(linv_hbm, linv_buf, sem_l, s, s, CH).start()
        _in_copy(adj_hbm, adj_buf, sem_a, s, s, CH).start()

    def outer(g, carry):
        for s in range(NBUF):
            i = g * NBUF + s
            _in_copy(linv_hbm, linv_buf, sem_l, i, s, CH).wait()
            _in_copy(adj_hbm, adj_buf, sem_a, i, s, CH).wait()

            h = jnp.maximum(adj_buf[s] * linv_buf[s], 0.0)
            res = jax.lax.dot_general(
                w, h, (((1,), (1,)), ((), ())),
                preferred_element_type=jnp.float32,
            ) + bias

            @pl.when(g > 0)
            def _wait_out():
                _out_copy(out_hbm, outbuf, sem_o, i - NBUF, s).wait()

            outbuf[s] = res

            @pl.when(i + NBUF < nchunk)
            def _next_in():
                _in_copy(linv_hbm, linv_buf, sem_l, i + NBUF, s, CH).start()
                _in_copy(adj_hbm, adj_buf, sem_a, i + NBUF, s, CH).start()

            _out_copy(out_hbm, outbuf, sem_o, i, s).start()
        return carry

    jax.lax.fori_loop(0, nchunk // NBUF, outer, 0)

    for s in range(NBUF):
        _out_copy(out_hbm, outbuf, sem_o, nchunk - NBUF + s, s).wait()


def _tc_part(Linv, adjacency, W, b2d, n_rows):
    nchunk = n_rows // CH
    out = pl.pallas_call(
        functools.partial(_tc_body, nchunk),
        in_specs=[
            pl.BlockSpec(memory_space=pltpu.MemorySpace.HBM),
            pl.BlockSpec(memory_space=pltpu.MemorySpace.HBM),
            pl.BlockSpec(memory_space=pltpu.MemorySpace.VMEM),
            pl.BlockSpec(memory_space=pltpu.MemorySpace.VMEM),
        ],
        out_specs=pl.BlockSpec(memory_space=pltpu.MemorySpace.HBM),
        out_shape=jax.ShapeDtypeStruct((nchunk, CH), jnp.float32),
        scratch_shapes=[
            pltpu.VMEM((NBUF, CH, C), jnp.float32),
            pltpu.VMEM((NBUF, CH, C), jnp.float32),
            pltpu.VMEM((NBUF, 1, CH), jnp.float32),
            pltpu.SemaphoreType.DMA((NBUF,)),
            pltpu.SemaphoreType.DMA((NBUF,)),
            pltpu.SemaphoreType.DMA((NBUF,)),
        ],
    )(Linv, adjacency, W, b2d)
    return out.reshape(n_rows)


NC = 2
NS = 16
NW = NC * NS
CHS = 80


def _sc_in_copy(hbm_ref, buf_ref, sem_ref, base, ch, chs):
    return pltpu.make_async_copy(
        hbm_ref.at[pl.ds(base + ch * chs, chs), :], buf_ref, sem_ref)


def _sc_compute_chunk(lbuf, abuf, w_v, b_v, out_v, out_base):
    bias = b_v[...]
    wsl = [w_v[pl.ds(16 * j, 16)] for j in range(C // 16)]
    mask15 = lax.iota(jnp.int32, 16) == 15

    @plsc.parallel_loop(0, CHS)
    def row(r):
        hs = []
        for j in range(C // 16):
            a = abuf[r, pl.ds(16 * j, 16)]
            l = lbuf[r, pl.ds(16 * j, 16)]
            hs.append(jnp.maximum(a * l, 0.0) * wsl[j])
        while len(hs) > 1:
            hs = [hs[k] + hs[k + 1] for k in range(0, len(hs) - 1, 2)] + (
                [hs[-1]] if len(hs) % 2 else [])
        tot = plsc.cumsum(hs[0]) + bias
        idx = jnp.full((16,), out_base + r, jnp.int32)
        plsc.store_scatter(out_v, [idx], tot, mask=mask15)


def _sc_part(Linv_sc, adjacency, W, b16, n_rows, row_off):
    rw = n_rows // NW
    nch = rw // CHS
    mesh = plsc.VectorSubcoreMesh(core_axis_name="c", subcore_axis_name="s")

    @functools.partial(
        pl.kernel, mesh=mesh,
        compiler_params=pltpu.CompilerParams(needs_layout_passes=False),
        out_type=jax.ShapeDtypeStruct((n_rows,), jnp.float32),
        scratch_types=[
            pltpu.VMEM((CHS, C), jnp.float32),
            pltpu.VMEM((CHS, C), jnp.float32),
            pltpu.VMEM((CHS, C), jnp.float32),
            pltpu.VMEM((CHS, C), jnp.float32),
            pltpu.VMEM((rw,), jnp.float32),
            pltpu.VMEM((C,), jnp.float32),
            pltpu.VMEM((16,), jnp.float32),
            pltpu.SemaphoreType.DMA,
            pltpu.SemaphoreType.DMA,
            pltpu.SemaphoreType.DMA,
            pltpu.SemaphoreType.DMA,
        ],
    )
    def sck(linv_hbm, adj_hbm, w_hbm, b_hbm, out_hbm,
            lbuf0, lbuf1, abuf0, abuf1, out_v, w_v, b_v,
            sl0, sl1, sa0, sa1):
        wid = lax.axis_index("s") * NC + lax.axis_index("c")
        base = row_off + wid * rw

        pltpu.sync_copy(w_hbm, w_v)
        pltpu.sync_copy(b_hbm, b_v)

        _sc_in_copy(linv_hbm, lbuf0, sl0, base, 0, CHS).start()
        _sc_in_copy(adj_hbm, abuf0, sa0, base, 0, CHS).start()
        _sc_in_copy(linv_hbm, lbuf1, sl1, base, 1, CHS).start()
        _sc_in_copy(adj_hbm, abuf1, sa1, base, 1, CHS).start()

        def pair(p, carry):
            ch0 = p * 2
            _sc_in_copy(linv_hbm, lbuf0, sl0, base, ch0, CHS).wait()
            _sc_in_copy(adj_hbm, abuf0, sa0, base, ch0, CHS).wait()
            _sc_compute_chunk(lbuf0, abuf0, w_v, b_v, out_v, ch0 * CHS)

            @pl.when(ch0 + 2 < nch)
            def _n0():
                _sc_in_copy(linv_hbm, lbuf0, sl0, base, ch0 + 2, CHS).start()
                _sc_in_copy(adj_hbm, abuf0, sa0, base, ch0 + 2, CHS).start()

            ch1 = ch0 + 1
            _sc_in_copy(linv_hbm, lbuf1, sl1, base, ch1, CHS).wait()
            _sc_in_copy(adj_hbm, abuf1, sa1, base, ch1, CHS).wait()
            _sc_compute_chunk(lbuf1, abuf1, w_v, b_v, out_v, ch1 * CHS)

            @pl.when(ch1 + 2 < nch)
            def _n1():
                _sc_in_copy(linv_hbm, lbuf1, sl1, base, ch1 + 2, CHS).start()
                _sc_in_copy(adj_hbm, abuf1, sa1, base, ch1 + 2, CHS).start()

            return carry

        lax.fori_loop(0, nch // 2, pair, 0)

        if nch % 2:
            ch = nch - 1
            _sc_in_copy(linv_hbm, lbuf0, sl0, base, ch, CHS).wait()
            _sc_in_copy(adj_hbm, abuf0, sa0, base, ch, CHS).wait()
            _sc_compute_chunk(lbuf0, abuf0, w_v, b_v, out_v, ch * CHS)

        pltpu.sync_copy(out_v, out_hbm.at[pl.ds(wid * rw, rw)])

    return sck(Linv_sc, adjacency, W.reshape(C), b16)


def kernel(x_e, Linv, adjacency, W, b):
    del x_e
    b16 = jnp.broadcast_to(b, (16,))
    out_sc = _sc_part(Linv, adjacency, W, b16, E, 0)
    return out_sc.reshape(E, 1)


# speedup vs baseline: 12.4064x; 1.3457x over previous
import functools

import jax
import jax.numpy as jnp
from jax import lax
from jax.experimental import pallas as pl
from jax.experimental.pallas import tpu as pltpu
from jax.experimental.pallas import tpu_sc as plsc

E = 320000
C = 128


CH = 2400
NBUF = 8


def _in_copy(hbm_ref, buf_ref, sem_ref, i, s, ch):
    return pltpu.make_async_copy(
        hbm_ref.at[pl.ds(i * ch, ch), :], buf_ref.at[s], sem_ref.at[s])


def _out_copy(out_hbm, outbuf, sem_ref, i, s):
    return pltpu.make_async_copy(
        outbuf.at[s], out_hbm.at[pl.ds(i, 1), :], sem_ref.at[s])


def _tc_body(nchunk, row_off, linv_hbm, adj_hbm, w_ref, b_ref, out_hbm,
             linv_buf, adj_buf, outbuf, sem_l, sem_a, sem_o):
    w = w_ref[...]
    bias = b_ref[0, 0]
    off = row_off // CH

    for s in range(NBUF):
        _in_copy(linv_hbm, linv_buf, sem_l, off + s, s, CH).start()
        _in_copy(adj_hbm, adj_buf, sem_a, off + s, s, CH).start()

    def outer(g, carry):
        for s in range(NBUF):
            i = g * NBUF + s
            _in_copy(linv_hbm, linv_buf, sem_l, off + i, s, CH).wait()
            _in_copy(adj_hbm, adj_buf, sem_a, off + i, s, CH).wait()

            h = jnp.maximum(adj_buf[s] * linv_buf[s], 0.0)
            res = jax.lax.dot_general(
                w, h, (((1,), (1,)), ((), ())),
                preferred_element_type=jnp.float32,
            ) + bias

            @pl.when(g > 0)
            def _wait_out():
                _out_copy(out_hbm, outbuf, sem_o, i - NBUF, s).wait()

            outbuf[s] = res

            @pl.when(i + NBUF < nchunk)
            def _next_in():
                _in_copy(linv_hbm, linv_buf, sem_l, off + i + NBUF, s, CH).start()
                _in_copy(adj_hbm, adj_buf, sem_a, off + i + NBUF, s, CH).start()

            _out_copy(out_hbm, outbuf, sem_o, i, s).start()
        return carry

    jax.lax.fori_loop(0, nchunk // NBUF, outer, 0)

    for s in range(NBUF):
        _out_copy(out_hbm, outbuf, sem_o, nchunk - NBUF + s, s).wait()


def _tc_part(Linv, adjacency, W, b2d, n_rows, row_off):
    nchunk = n_rows // CH
    out = pl.pallas_call(
        functools.partial(_tc_body, nchunk, row_off),
        in_specs=[
            pl.BlockSpec(memory_space=pltpu.MemorySpace.HBM),
            pl.BlockSpec(memory_space=pltpu.MemorySpace.HBM),
            pl.BlockSpec(memory_space=pltpu.MemorySpace.VMEM),
            pl.BlockSpec(memory_space=pltpu.MemorySpace.VMEM),
        ],
        out_specs=pl.BlockSpec(memory_space=pltpu.MemorySpace.HBM),
        out_shape=jax.ShapeDtypeStruct((nchunk, CH), jnp.float32),
        scratch_shapes=[
            pltpu.VMEM((NBUF, CH, C), jnp.float32),
            pltpu.VMEM((NBUF, CH, C), jnp.float32),
            pltpu.VMEM((NBUF, 1, CH), jnp.float32),
            pltpu.SemaphoreType.DMA((NBUF,)),
            pltpu.SemaphoreType.DMA((NBUF,)),
            pltpu.SemaphoreType.DMA((NBUF,)),
        ],
    )(Linv, adjacency, W, b2d)
    return out.reshape(n_rows)


NC = 2
NS = 16
NW = NC * NS
CHS = 80


def _sc_in_copy(hbm_ref, buf_ref, sem_ref, base, ch, chs):
    return pltpu.make_async_copy(
        hbm_ref.at[pl.ds(base + ch * chs, chs), :], buf_ref, sem_ref)


def _sc_compute_chunk(lbuf, abuf, w_v, b_v, out_v, out_base):
    bias = b_v[...]
    wsl = [w_v[pl.ds(16 * j, 16)] for j in range(C // 16)]
    mask15 = lax.iota(jnp.int32, 16) == 15

    @plsc.parallel_loop(0, CHS)
    def row(r):
        hs = []
        for j in range(C // 16):
            a = abuf[r, pl.ds(16 * j, 16)]
            l = lbuf[r, pl.ds(16 * j, 16)]
            hs.append(jnp.maximum(a * l, 0.0) * wsl[j])
        while len(hs) > 1:
            hs = [hs[k] + hs[k + 1] for k in range(0, len(hs) - 1, 2)] + (
                [hs[-1]] if len(hs) % 2 else [])
        tot = plsc.cumsum(hs[0]) + bias
        idx = jnp.full((16,), out_base + r, jnp.int32)
        plsc.store_scatter(out_v, [idx], tot, mask=mask15)


def _sc_part(Linv_sc, adjacency, W, b16, n_rows, row_off):
    rw = n_rows // NW
    nch = rw // CHS
    mesh = plsc.VectorSubcoreMesh(core_axis_name="c", subcore_axis_name="s")

    @functools.partial(
        pl.kernel, mesh=mesh,
        compiler_params=pltpu.CompilerParams(needs_layout_passes=False),
        out_type=jax.ShapeDtypeStruct((n_rows,), jnp.float32),
        scratch_types=[
            pltpu.VMEM((CHS, C), jnp.float32),
            pltpu.VMEM((CHS, C), jnp.float32),
            pltpu.VMEM((CHS, C), jnp.float32),
            pltpu.VMEM((CHS, C), jnp.float32),
            pltpu.VMEM((rw,), jnp.float32),
            pltpu.VMEM((C,), jnp.float32),
            pltpu.VMEM((16,), jnp.float32),
            pltpu.SemaphoreType.DMA,
            pltpu.SemaphoreType.DMA,
            pltpu.SemaphoreType.DMA,
            pltpu.SemaphoreType.DMA,
        ],
    )
    def sck(linv_hbm, adj_hbm, w_hbm, b_hbm, out_hbm,
            lbuf0, lbuf1, abuf0, abuf1, out_v, w_v, b_v,
            sl0, sl1, sa0, sa1):
        wid = lax.axis_index("s") * NC + lax.axis_index("c")
        base = row_off + wid * rw

        pltpu.sync_copy(w_hbm, w_v)
        pltpu.sync_copy(b_hbm, b_v)

        _sc_in_copy(linv_hbm, lbuf0, sl0, base, 0, CHS).start()
        _sc_in_copy(adj_hbm, abuf0, sa0, base, 0, CHS).start()
        _sc_in_copy(linv_hbm, lbuf1, sl1, base, 1, CHS).start()
        _sc_in_copy(adj_hbm, abuf1, sa1, base, 1, CHS).start()

        def pair(p, carry):
            ch0 = p * 2
            _sc_in_copy(linv_hbm, lbuf0, sl0, base, ch0, CHS).wait()
            _sc_in_copy(adj_hbm, abuf0, sa0, base, ch0, CHS).wait()
            _sc_compute_chunk(lbuf0, abuf0, w_v, b_v, out_v, ch0 * CHS)

            @pl.when(ch0 + 2 < nch)
            def _n0():
                _sc_in_copy(linv_hbm, lbuf0, sl0, base, ch0 + 2, CHS).start()
                _sc_in_copy(adj_hbm, abuf0, sa0, base, ch0 + 2, CHS).start()

            ch1 = ch0 + 1
            _sc_in_copy(linv_hbm, lbuf1, sl1, base, ch1, CHS).wait()
            _sc_in_copy(adj_hbm, abuf1, sa1, base, ch1, CHS).wait()
            _sc_compute_chunk(lbuf1, abuf1, w_v, b_v, out_v, ch1 * CHS)

            @pl.when(ch1 + 2 < nch)
            def _n1():
                _sc_in_copy(linv_hbm, lbuf1, sl1, base, ch1 + 2, CHS).start()
                _sc_in_copy(adj_hbm, abuf1, sa1, base, ch1 + 2, CHS).start()

            return carry

        lax.fori_loop(0, nch // 2, pair, 0)

        if nch % 2:
            ch = nch - 1
            _sc_in_copy(linv_hbm, lbuf0, sl0, base, ch, CHS).wait()
            _sc_in_copy(adj_hbm, abuf0, sa0, base, ch, CHS).wait()
            _sc_compute_chunk(lbuf0, abuf0, w_v, b_v, out_v, ch * CHS)

        pltpu.sync_copy(out_v, out_hbm.at[pl.ds(wid * rw, rw)])

    return sck(Linv_sc, adjacency, W.reshape(C), b16)


E_SC = 128000


def kernel(x_e, Linv, adjacency, W, b):
    del x_e
    b16 = jnp.broadcast_to(b, (16,))
    out_sc = _sc_part(Linv, adjacency, W, b16, E_SC, 0)
    out_tc = _tc_part(Linv, adjacency, W, b.reshape(1, 1), E - E_SC, E_SC)
    return jnp.concatenate([out_sc, out_tc]).reshape(E, 1)


# speedup vs baseline: 12.5290x; 1.0099x over previous
import functools

import jax
import jax.numpy as jnp
from jax import lax
from jax.experimental import pallas as pl
from jax.experimental.pallas import tpu as pltpu
from jax.experimental.pallas import tpu_sc as plsc

E = 320000
C = 128


CH = 2400
NBUF = 8


def _in_copy(hbm_ref, buf_ref, sem_ref, row0, s, ch):
    return pltpu.make_async_copy(
        hbm_ref.at[pl.ds(row0, ch), :], buf_ref.at[s], sem_ref.at[s])


def _out_copy(out_hbm, outbuf, sem_ref, i, s):
    return pltpu.make_async_copy(
        outbuf.at[s], out_hbm.at[pl.ds(i, 1), :], sem_ref.at[s])


def _tc_body(nchunk, row_off, linv_hbm, adj_hbm, w_ref, b_ref, out_hbm,
             linv_buf, adj_buf, outbuf, sem_l, sem_a, sem_o):
    w = w_ref[...]
    bias = b_ref[0, 0]
    for s in range(NBUF):
        _in_copy(linv_hbm, linv_buf, sem_l, row_off + s * CH, s, CH).start()
        _in_copy(adj_hbm, adj_buf, sem_a, row_off + s * CH, s, CH).start()

    def outer(g, carry):
        for s in range(NBUF):
            i = g * NBUF + s
            _in_copy(linv_hbm, linv_buf, sem_l, row_off + i * CH, s, CH).wait()
            _in_copy(adj_hbm, adj_buf, sem_a, row_off + i * CH, s, CH).wait()

            h = jnp.maximum(adj_buf[s] * linv_buf[s], 0.0)
            res = jax.lax.dot_general(
                w, h, (((1,), (1,)), ((), ())),
                preferred_element_type=jnp.float32,
            ) + bias

            @pl.when(g > 0)
            def _wait_out():
                _out_copy(out_hbm, outbuf, sem_o, i - NBUF, s).wait()

            outbuf[s] = res

            @pl.when(i + NBUF < nchunk)
            def _next_in():
                _in_copy(linv_hbm, linv_buf, sem_l,
                         row_off + (i + NBUF) * CH, s, CH).start()
                _in_copy(adj_hbm, adj_buf, sem_a,
                         row_off + (i + NBUF) * CH, s, CH).start()

            _out_copy(out_hbm, outbuf, sem_o, i, s).start()
        return carry

    jax.lax.fori_loop(0, nchunk // NBUF, outer, 0)

    for s in range(NBUF):
        _out_copy(out_hbm, outbuf, sem_o, nchunk - NBUF + s, s).wait()


def _tc_part(Linv, adjacency, W, b2d, n_rows, row_off):
    nchunk = n_rows // CH
    out = pl.pallas_call(
        functools.partial(_tc_body, nchunk, row_off),
        in_specs=[
            pl.BlockSpec(memory_space=pltpu.MemorySpace.HBM),
            pl.BlockSpec(memory_space=pltpu.MemorySpace.HBM),
            pl.BlockSpec(memory_space=pltpu.MemorySpace.VMEM),
            pl.BlockSpec(memory_space=pltpu.MemorySpace.VMEM),
        ],
        out_specs=pl.BlockSpec(memory_space=pltpu.MemorySpace.HBM),
        out_shape=jax.ShapeDtypeStruct((nchunk, CH), jnp.float32),
        scratch_shapes=[
            pltpu.VMEM((NBUF, CH, C), jnp.float32),
            pltpu.VMEM((NBUF, CH, C), jnp.float32),
            pltpu.VMEM((NBUF, 1, CH), jnp.float32),
            pltpu.SemaphoreType.DMA((NBUF,)),
            pltpu.SemaphoreType.DMA((NBUF,)),
            pltpu.SemaphoreType.DMA((NBUF,)),
        ],
    )(Linv, adjacency, W, b2d)
    return out.reshape(n_rows)


NC = 2
NS = 16
NW = NC * NS
CHS = 80


def _sc_in_copy(hbm_ref, buf_ref, sem_ref, base, ch, chs):
    return pltpu.make_async_copy(
        hbm_ref.at[pl.ds(base + ch * chs, chs), :], buf_ref, sem_ref)


def _sc_compute_chunk(lbuf, abuf, w_v, b_v, out_v, out_base):
    bias = b_v[...]
    wsl = [w_v[pl.ds(16 * j, 16)] for j in range(C // 16)]
    mask15 = lax.iota(jnp.int32, 16) == 15

    @plsc.parallel_loop(0, CHS)
    def row(r):
        hs = []
        for j in range(C // 16):
            a = abuf[r, pl.ds(16 * j, 16)]
            l = lbuf[r, pl.ds(16 * j, 16)]
            hs.append(jnp.maximum(a * l, 0.0) * wsl[j])
        while len(hs) > 1:
            hs = [hs[k] + hs[k + 1] for k in range(0, len(hs) - 1, 2)] + (
                [hs[-1]] if len(hs) % 2 else [])
        tot = plsc.cumsum(hs[0]) + bias
        idx = jnp.full((16,), out_base + r, jnp.int32)
        plsc.store_scatter(out_v, [idx], tot, mask=mask15)


def _sc_part(Linv_sc, adjacency, W, b16, n_rows, row_off):
    rw = n_rows // NW
    nch = rw // CHS
    mesh = plsc.VectorSubcoreMesh(core_axis_name="c", subcore_axis_name="s")

    @functools.partial(
        pl.kernel, mesh=mesh,
        compiler_params=pltpu.CompilerParams(needs_layout_passes=False),
        out_type=jax.ShapeDtypeStruct((n_rows,), jnp.float32),
        scratch_types=[
            pltpu.VMEM((CHS, C), jnp.float32),
            pltpu.VMEM((CHS, C), jnp.float32),
            pltpu.VMEM((CHS, C), jnp.float32),
            pltpu.VMEM((CHS, C), jnp.float32),
            pltpu.VMEM((rw,), jnp.float32),
            pltpu.VMEM((C,), jnp.float32),
            pltpu.VMEM((16,), jnp.float32),
            pltpu.SemaphoreType.DMA,
            pltpu.SemaphoreType.DMA,
            pltpu.SemaphoreType.DMA,
            pltpu.SemaphoreType.DMA,
        ],
    )
    def sck(linv_hbm, adj_hbm, w_hbm, b_hbm, out_hbm,
            lbuf0, lbuf1, abuf0, abuf1, out_v, w_v, b_v,
            sl0, sl1, sa0, sa1):
        wid = lax.axis_index("s") * NC + lax.axis_index("c")
        base = row_off + wid * rw

        pltpu.sync_copy(w_hbm, w_v)
        pltpu.sync_copy(b_hbm, b_v)

        _sc_in_copy(linv_hbm, lbuf0, sl0, base, 0, CHS).start()
        _sc_in_copy(adj_hbm, abuf0, sa0, base, 0, CHS).start()
        _sc_in_copy(linv_hbm, lbuf1, sl1, base, 1, CHS).start()
        _sc_in_copy(adj_hbm, abuf1, sa1, base, 1, CHS).start()

        def pair(p, carry):
            ch0 = p * 2
            _sc_in_copy(linv_hbm, lbuf0, sl0, base, ch0, CHS).wait()
            _sc_in_copy(adj_hbm, abuf0, sa0, base, ch0, CHS).wait()
            _sc_compute_chunk(lbuf0, abuf0, w_v, b_v, out_v, ch0 * CHS)

            @pl.when(ch0 + 2 < nch)
            def _n0():
                _sc_in_copy(linv_hbm, lbuf0, sl0, base, ch0 + 2, CHS).start()
                _sc_in_copy(adj_hbm, abuf0, sa0, base, ch0 + 2, CHS).start()

            ch1 = ch0 + 1
            _sc_in_copy(linv_hbm, lbuf1, sl1, base, ch1, CHS).wait()
            _sc_in_copy(adj_hbm, abuf1, sa1, base, ch1, CHS).wait()
            _sc_compute_chunk(lbuf1, abuf1, w_v, b_v, out_v, ch1 * CHS)

            @pl.when(ch1 + 2 < nch)
            def _n1():
                _sc_in_copy(linv_hbm, lbuf1, sl1, base, ch1 + 2, CHS).start()
                _sc_in_copy(adj_hbm, abuf1, sa1, base, ch1 + 2, CHS).start()

            return carry

        lax.fori_loop(0, nch // 2, pair, 0)

        if nch % 2:
            ch = nch - 1
            _sc_in_copy(linv_hbm, lbuf0, sl0, base, ch, CHS).wait()
            _sc_in_copy(adj_hbm, abuf0, sa0, base, ch, CHS).wait()
            _sc_compute_chunk(lbuf0, abuf0, w_v, b_v, out_v, ch * CHS)

        pltpu.sync_copy(out_v, out_hbm.at[pl.ds(wid * rw, rw)])

    return sck(Linv_sc, adjacency, W.reshape(C), b16)


E_SC = 128000


def kernel(x_e, Linv, adjacency, W, b):
    del x_e
    b16 = jnp.broadcast_to(b, (16,))
    out_sc = _sc_part(Linv, adjacency, W, b16, E_SC, 0)
    out_tc = _tc_part(Linv, adjacency, W, b.reshape(1, 1), E - E_SC, E_SC)
    return jnp.concatenate([out_sc, out_tc]).reshape(E, 1)
